# baseline (speedup 1.0000x reference)
import jax
import jax.numpy as jnp
from jax.experimental import pallas as pl

_MAXLEN = 8192
_EMBED = 128
_BATCH = 4
_TBLK = 4096


def _add_body(x_ref, p_ref, o_ref):
    o_ref[...] = x_ref[...] + p_ref[...][None, :, :]


def kernel(x, pos_table):
    return pl.pallas_call(
        _add_body,
        grid=(_MAXLEN // _TBLK,),
        in_specs=[
            pl.BlockSpec((_BATCH, _TBLK, _EMBED), lambda t: (0, t, 0)),
            pl.BlockSpec((_TBLK, _EMBED), lambda t: (t, 0)),
        ],
        out_specs=pl.BlockSpec((_BATCH, _TBLK, _EMBED), lambda t: (0, t, 0)),
        out_shape=jax.ShapeDtypeStruct((_BATCH, _MAXLEN, _EMBED), jnp.float32),
    )(x, pos_table)
